# baseline (speedup 1.0000x reference)
import functools

import jax
import jax.numpy as jnp
from jax import lax
from jax.experimental import pallas as pl
from jax.experimental.pallas import tpu as pltpu
from jax.experimental.pallas import tpu_sc as plsc

N_EMB = 8192
DIM = 256
N_TOK = 16384

TB = 256
CB = 2816
NJ = 3
N_PAD = NJ * CB


def _argmin_body(a_ref, se_ref, f_ref, e_ref, idx_ref, best_ref, bidx_ref):
    j = pl.program_id(0)
    t = pl.program_id(1)
    f = f_ref[...].astype(jnp.bfloat16)
    sim2 = jnp.dot(f, e_ref[...], preferred_element_type=jnp.float32)
    dist = (a_ref[...] + se_ref[...]) - sim2
    m = jnp.min(dist, axis=1, keepdims=True)
    lane = lax.broadcasted_iota(jnp.int32, dist.shape, 1)
    cand = jnp.min(jnp.where(dist == m, lane, jnp.int32(N_PAD)),
                   axis=1, keepdims=True) + j * CB
    mr = m.astype(jnp.bfloat16).astype(jnp.float32)
    sl = pl.ds(t * TB, TB)

    @pl.when(j == 0)
    def _():
        best_ref[sl, :] = mr
        bidx_ref[sl, :] = cand

    @pl.when(j > 0)
    def _():
        keep = best_ref[sl, :] <= m
        bidx_ref[sl, :] = jnp.where(keep, bidx_ref[sl, :], cand)
        best_ref[sl, :] = jnp.where(keep, best_ref[sl, :], mr)

    @pl.when(j == NJ - 1)
    def _():
        idx_ref[...] = bidx_ref[sl, :][:, 0]


def _compute_indices(f, e2b, a, se):
    n = f.shape[0]
    return pl.pallas_call(
        _argmin_body,
        grid=(NJ, n // TB),
        in_specs=[
            pl.BlockSpec((TB, 1), lambda j, t: (t, 0)),
            pl.BlockSpec((1, CB), lambda j, t: (0, j)),
            pl.BlockSpec((TB, DIM), lambda j, t: (t, 0)),
            pl.BlockSpec((DIM, CB), lambda j, t: (0, j)),
        ],
        out_specs=pl.BlockSpec((TB,), lambda j, t: (t,)),
        out_shape=jax.ShapeDtypeStruct((n,), jnp.int32),
        scratch_shapes=[
            pltpu.VMEM((n, 1), jnp.float32),
            pltpu.VMEM((n, 1), jnp.int32),
        ],
        compiler_params=pltpu.CompilerParams(
            dimension_semantics=("arbitrary", "arbitrary"),
        ),
    )(a, se, f, e2b)


_NC = 2
_NS = 16
_NW = _NC * _NS
_CHUNK = 128


def _gather_st(table, idx, xflat):
    n = idx.shape[0]
    _BPW = n // _NW
    _NCH = _BPW // _CHUNK
    mesh = plsc.VectorSubcoreMesh(core_axis_name="c", subcore_axis_name="s")

    @functools.partial(
        pl.kernel,
        mesh=mesh,
        out_type=jax.ShapeDtypeStruct((n, DIM), jnp.float32),
        scratch_types=[
            pltpu.VMEM((_BPW,), jnp.int32),
            pltpu.VMEM((_CHUNK, DIM), jnp.float32),
            pltpu.VMEM((_CHUNK, DIM), jnp.float32),
            pltpu.SemaphoreType.DMA,
        ],
    )
    def k(table_hbm, idx_hbm, x_hbm, out_hbm, idx_v, rows_v, x_v, sem):
        wid = lax.axis_index("s") * _NC + lax.axis_index("c")
        base = wid * _BPW
        pltpu.sync_copy(idx_hbm.at[pl.ds(base, _BPW)], idx_v)
        for c in range(_NCH):
            pltpu.async_copy(
                table_hbm.at[idx_v.at[pl.ds(c * _CHUNK, _CHUNK)]],
                rows_v, sem).wait()
            pltpu.sync_copy(x_hbm.at[pl.ds(base + c * _CHUNK, _CHUNK)], x_v)

            def row(r, _):
                def vec(v, _):
                    xv = x_v[r, pl.ds(v * 16, 16)]
                    qv = rows_v[r, pl.ds(v * 16, 16)]
                    rows_v[r, pl.ds(v * 16, 16)] = xv + (qv - xv)
                    return 0
                lax.fori_loop(0, DIM // 16, vec, 0)
                return 0
            lax.fori_loop(0, _CHUNK, row, 0)
            pltpu.sync_copy(rows_v, out_hbm.at[pl.ds(base + c * _CHUNK, _CHUNK)])

    return k(table, idx, xflat)


def kernel(x, embeddings):
    f = x.reshape(-1, DIM)
    a = jnp.sum(f ** 2, axis=1, keepdims=True)
    se = jnp.sum(embeddings ** 2, axis=0, keepdims=True)
    e_pad = jnp.pad(embeddings, ((0, 0), (0, N_PAD - N_EMB)))
    e2b = (2.0 * e_pad).astype(jnp.bfloat16)
    se_pad = jnp.concatenate(
        [se, jnp.full((1, N_PAD - N_EMB), jnp.inf, jnp.float32)], axis=1)
    table = embeddings.T
    h = N_TOK // 2
    outs = []
    for lo in (0, h):
        fi = f[lo:lo + h]
        idx = _compute_indices(fi, e2b, a[lo:lo + h], se_pad)
        outs.append(_gather_st(table, idx, fi))
    return jnp.concatenate(outs).reshape(x.shape)


# speedup vs baseline: 1.1378x; 1.1378x over previous
import functools

import jax
import jax.numpy as jnp
from jax import lax
from jax.experimental import pallas as pl
from jax.experimental.pallas import tpu as pltpu
from jax.experimental.pallas import tpu_sc as plsc

N_EMB = 8192
DIM = 256
N_TOK = 16384

TB = 512
CB = 2816
NJ = 3
N_PAD = NJ * CB


def _argmin_body(a_ref, se_ref, f_ref, e_ref, idx_ref, best_ref, bidx_ref):
    j = pl.program_id(0)
    t = pl.program_id(1)
    f = f_ref[...].astype(jnp.bfloat16)
    sim2 = jnp.dot(f, e_ref[...], preferred_element_type=jnp.float32)
    dist = (a_ref[...] + se_ref[...]) - sim2
    m = jnp.min(dist, axis=1, keepdims=True)
    lane = lax.broadcasted_iota(jnp.int32, dist.shape, 1)
    cand = jnp.min(jnp.where(dist == m, lane, jnp.int32(N_PAD)),
                   axis=1, keepdims=True) + j * CB
    mr = m.astype(jnp.bfloat16).astype(jnp.float32)
    sl = pl.ds(t * TB, TB)

    @pl.when(j == 0)
    def _():
        best_ref[sl, :] = mr
        bidx_ref[sl, :] = cand

    @pl.when(j > 0)
    def _():
        keep = best_ref[sl, :] <= m
        bidx_ref[sl, :] = jnp.where(keep, bidx_ref[sl, :], cand)
        best_ref[sl, :] = jnp.where(keep, best_ref[sl, :], mr)

    @pl.when(j == NJ - 1)
    def _():
        idx_ref[...] = bidx_ref[sl, :][:, 0]


def _compute_indices(f, e2b, a, se):
    n = f.shape[0]
    return pl.pallas_call(
        _argmin_body,
        grid=(NJ, n // TB),
        in_specs=[
            pl.BlockSpec((TB, 1), lambda j, t: (t, 0)),
            pl.BlockSpec((1, CB), lambda j, t: (0, j)),
            pl.BlockSpec((TB, DIM), lambda j, t: (t, 0)),
            pl.BlockSpec((DIM, CB), lambda j, t: (0, j)),
        ],
        out_specs=pl.BlockSpec((TB,), lambda j, t: (t,)),
        out_shape=jax.ShapeDtypeStruct((n,), jnp.int32),
        scratch_shapes=[
            pltpu.VMEM((n, 1), jnp.float32),
            pltpu.VMEM((n, 1), jnp.int32),
        ],
        compiler_params=pltpu.CompilerParams(
            dimension_semantics=("arbitrary", "arbitrary"),
        ),
    )(a, se, f, e2b)


_NC = 2
_NS = 16
_NW = _NC * _NS
_CHUNK = 128


def _gather_st(table, idx, xflat):
    n = idx.shape[0]
    _BPW = n // _NW
    _NCH = _BPW // _CHUNK
    mesh = plsc.VectorSubcoreMesh(core_axis_name="c", subcore_axis_name="s")

    @functools.partial(
        pl.kernel,
        mesh=mesh,
        out_type=jax.ShapeDtypeStruct((n, DIM), jnp.float32),
        scratch_types=[
            pltpu.VMEM((_BPW,), jnp.int32),
            pltpu.VMEM((_CHUNK, DIM), jnp.float32),
            pltpu.VMEM((_CHUNK, DIM), jnp.float32),
            pltpu.SemaphoreType.DMA,
        ],
    )
    def k(table_hbm, idx_hbm, x_hbm, out_hbm, idx_v, rows_v, x_v, sem):
        wid = lax.axis_index("s") * _NC + lax.axis_index("c")
        base = wid * _BPW
        pltpu.sync_copy(idx_hbm.at[pl.ds(base, _BPW)], idx_v)
        for c in range(_NCH):
            pltpu.async_copy(
                table_hbm.at[idx_v.at[pl.ds(c * _CHUNK, _CHUNK)]],
                rows_v, sem).wait()
            pltpu.sync_copy(x_hbm.at[pl.ds(base + c * _CHUNK, _CHUNK)], x_v)

            def row(r, _):
                def vec(v, _):
                    xv = x_v[r, pl.ds(v * 16, 16)]
                    qv = rows_v[r, pl.ds(v * 16, 16)]
                    rows_v[r, pl.ds(v * 16, 16)] = xv + (qv - xv)
                    return 0
                lax.fori_loop(0, DIM // 16, vec, 0)
                return 0
            lax.fori_loop(0, _CHUNK, row, 0)
            pltpu.sync_copy(rows_v, out_hbm.at[pl.ds(base + c * _CHUNK, _CHUNK)])

    return k(table, idx, xflat)


def kernel(x, embeddings):
    f = x.reshape(-1, DIM)
    a = jnp.sum(f ** 2, axis=1, keepdims=True)
    se = jnp.sum(embeddings ** 2, axis=0, keepdims=True)
    e_pad = jnp.pad(embeddings, ((0, 0), (0, N_PAD - N_EMB)))
    e2b = (2.0 * e_pad).astype(jnp.bfloat16)
    se_pad = jnp.concatenate(
        [se, jnp.full((1, N_PAD - N_EMB), jnp.inf, jnp.float32)], axis=1)
    table = embeddings.T
    h = N_TOK // 2
    outs = []
    for lo in (0, h):
        fi = f[lo:lo + h]
        idx = _compute_indices(fi, e2b, a[lo:lo + h], se_pad)
        outs.append(_gather_st(table, idx, fi))
    return jnp.concatenate(outs).reshape(x.shape)
